# Initial kernel scaffold; baseline (speedup 1.0000x reference)
import functools

import jax
import jax.numpy as jnp
from jax import lax
from jax.experimental import pallas as pl
from jax.experimental.pallas import tpu as pltpu
from jax.experimental.pallas import tpu_sc as plsc

N = 10000
C = 128
E = 320000
NNZ = 320000
NHE = 2000
K = 80
NSUB = 16
CL = 16

_mesh = lambda: plsc.VectorSubcoreMesh(core_axis_name="c", subcore_axis_name="s")


def _mlp_body(x_ref, w1_ref, b1_ref, w2_ref, b2_ref, o_ref):
    z = jnp.dot(x_ref[...], w1_ref[...], preferred_element_type=jnp.float32)
    z = z + b1_ref[...]
    z = jnp.where(z >= 0.0, z, 0.2 * z)
    o_ref[...] = jnp.dot(z, w2_ref[...], preferred_element_type=jnp.float32) + b2_ref[...]


def _mlp(x, W1, b1, W2, b2):
    R = 1000
    return pl.pallas_call(
        _mlp_body,
        grid=(N // R,),
        in_specs=[
            pl.BlockSpec((R, C), lambda i: (i, 0)),
            pl.BlockSpec((C, C // 2), lambda i: (0, 0)),
            pl.BlockSpec((1, C // 2), lambda i: (0, 0)),
            pl.BlockSpec((C // 2, C), lambda i: (0, 0)),
            pl.BlockSpec((1, C), lambda i: (0, 0)),
        ],
        out_specs=pl.BlockSpec((R, C), lambda i: (i, 0)),
        out_shape=jax.ShapeDtypeStruct((N, C), jnp.float32),
    )(x, W1, b1, W2, b2)


def _sc_pass(s, table_hbm, gidx_hbm, sidx_hbm, z128_hbm, z16_hbm, ones_hbm,
             sum_out, cnt_out, gidx_v, sidx_v, rows_v, ones_v, acc_sh, cnt_sh,
             rows_per_tile, base0, nchunks):
    pltpu.sync_copy(z128_hbm.at[pl.ds(0, rows_per_tile)],
                    acc_sh.at[pl.ds(s * rows_per_tile, rows_per_tile)])
    pltpu.sync_copy(z16_hbm.at[pl.ds(0, rows_per_tile)],
                    cnt_sh.at[pl.ds(s * rows_per_tile, rows_per_tile)])
    pltpu.sync_copy(ones_hbm, ones_v)
    plsc.subcore_barrier()

    def chunk(i, carry):
        base = base0 + i * K
        pltpu.sync_copy(gidx_hbm.at[pl.ds(base, K)], gidx_v)
        pltpu.sync_copy(sidx_hbm.at[pl.ds(base, K)], sidx_v)
        pltpu.sync_copy(table_hbm.at[gidx_v], rows_v)
        pltpu.sync_copy(rows_v, acc_sh.at[sidx_v], add=True)
        pltpu.sync_copy(ones_v, cnt_sh.at[sidx_v], add=True)
        return carry

    lax.fori_loop(0, nchunks, chunk, 0)
    plsc.subcore_barrier()
    pltpu.sync_copy(acc_sh.at[pl.ds(s * rows_per_tile, rows_per_tile)],
                    sum_out.at[pl.ds(s * rows_per_tile, rows_per_tile)])
    pltpu.sync_copy(cnt_sh.at[pl.ds(s * rows_per_tile, rows_per_tile)],
                    cnt_out.at[pl.ds(s * rows_per_tile, rows_per_tile)])


def _scA_body(h_hbm, src_hbm, dst_hbm, hgv_hbm, hge_hbm, z128_hbm, z16_hbm,
              ones_hbm, gsum_out, gcnt_out, esum_out, ecnt_out,
              gidx_v, sidx_v, rows_v, ones_v, acc_sh, cnt_sh):
    c = lax.axis_index("c")
    s = lax.axis_index("s")
    per_tile = E // NSUB

    @pl.when(c == 0)
    def _():
        _sc_pass(s, h_hbm, src_hbm, dst_hbm, z128_hbm, z16_hbm, ones_hbm,
                 gsum_out, gcnt_out, gidx_v, sidx_v, rows_v, ones_v,
                 acc_sh, cnt_sh, N // NSUB, s * per_tile, per_tile // K)

    @pl.when(c == 1)
    def _():
        _sc_pass(s, h_hbm, hgv_hbm, hge_hbm, z128_hbm, z16_hbm, ones_hbm,
                 esum_out, ecnt_out, gidx_v, sidx_v, rows_v, ones_v,
                 acc_sh, cnt_sh, NHE // NSUB, s * per_tile, per_tile // K)


def _scA(h, src, dst, hgv, hge, z128, z16, ones):
    f = pl.kernel(
        _scA_body,
        out_type=[
            jax.ShapeDtypeStruct((N, C), jnp.float32),
            jax.ShapeDtypeStruct((N, CL), jnp.float32),
            jax.ShapeDtypeStruct((NHE, C), jnp.float32),
            jax.ShapeDtypeStruct((NHE, CL), jnp.float32),
        ],
        mesh=_mesh(),
        scratch_types=[
            pltpu.VMEM((K,), jnp.int32),
            pltpu.VMEM((K,), jnp.int32),
            pltpu.VMEM((K, C), jnp.float32),
            pltpu.VMEM((K, CL), jnp.float32),
            pltpu.VMEM_SHARED((N, C), jnp.float32),
            pltpu.VMEM_SHARED((N, CL), jnp.float32),
        ],
    )
    return f(h, src, dst, hgv, hge, z128, z16, ones)


def _scB_body(ef_hbm, hge_hbm, hgv_hbm, z128_hbm, z16_hbm, ones_hbm,
              nsum_out, ncnt_out, gidx_v, sidx_v, rows_v, ones_v, acc_sh, cnt_sh):
    c = lax.axis_index("c")
    s = lax.axis_index("s")
    wid = s * 2 + c
    per_w = NNZ // (2 * NSUB)
    _sc_pass(s, ef_hbm, hge_hbm, hgv_hbm, z128_hbm, z16_hbm, ones_hbm,
             nsum_out.at[c], ncnt_out.at[c], gidx_v, sidx_v, rows_v, ones_v,
             acc_sh, cnt_sh, N // NSUB, wid * per_w, per_w // K)


def _scB(ef, hge, hgv, z128, z16, ones):
    f = pl.kernel(
        _scB_body,
        out_type=[
            jax.ShapeDtypeStruct((2, N, C), jnp.float32),
            jax.ShapeDtypeStruct((2, N, CL), jnp.float32),
        ],
        mesh=_mesh(),
        scratch_types=[
            pltpu.VMEM((K,), jnp.int32),
            pltpu.VMEM((K,), jnp.int32),
            pltpu.VMEM((K, C), jnp.float32),
            pltpu.VMEM((K, CL), jnp.float32),
            pltpu.VMEM_SHARED((N, C), jnp.float32),
            pltpu.VMEM_SHARED((N, CL), jnp.float32),
        ],
    )
    return f(ef, hge, hgv, z128, z16, ones)


def _ecomb_body(es_ref, ec_ref, o_ref):
    cnt = jnp.maximum(ec_ref[:, 0:1], 1.0)
    o_ref[...] = es_ref[...] / cnt


def _ecomb(esum, ecnt):
    return pl.pallas_call(
        _ecomb_body,
        out_shape=jax.ShapeDtypeStruct((NHE, C), jnp.float32),
    )(esum, ecnt)


def _final_body(wv_ref, h_ref, gs_ref, gc_ref, ns_ref, nc_ref, o_ref):
    w1 = wv_ref[0, 0]
    w2 = wv_ref[0, 1]
    xg = gs_ref[...] / jnp.maximum(gc_ref[:, 0:1], 1.0)
    nsum = ns_ref[0] + ns_ref[1]
    ncnt = nc_ref[0, :, 0:1] + nc_ref[1, :, 0:1]
    xhg = nsum / jnp.maximum(ncnt, 1.0)
    out = w1 * ((xg + xhg) * 0.5) + w2 * h_ref[...]
    o_ref[...] = jnp.where(out >= 0.0, out, 0.2 * out)


def _final(wv, h, gsum, gcnt, nsum, ncnt):
    R = 1000
    return pl.pallas_call(
        _final_body,
        grid=(N // R,),
        in_specs=[
            pl.BlockSpec(memory_space=pltpu.SMEM),
            pl.BlockSpec((R, C), lambda i: (i, 0)),
            pl.BlockSpec((R, C), lambda i: (i, 0)),
            pl.BlockSpec((R, CL), lambda i: (i, 0)),
            pl.BlockSpec((2, R, C), lambda i: (0, i, 0)),
            pl.BlockSpec((2, R, CL), lambda i: (0, i, 0)),
        ],
        out_specs=pl.BlockSpec((R, C), lambda i: (i, 0)),
        out_shape=jax.ShapeDtypeStruct((N, C), jnp.float32),
    )(wv, h, gsum, gcnt, nsum, ncnt)


def kernel(x, w, W1, b1, W2, b2, graph_edge_index, hg_vertex, hg_edge):
    src = graph_edge_index[0]
    dst = graph_edge_index[1]
    ew = jnp.exp(w)
    wv = (ew / jnp.sum(ew)).reshape(1, 2)
    z128 = jnp.zeros((N // NSUB, C), jnp.float32)
    z16 = jnp.zeros((N // NSUB, CL), jnp.float32)
    ones = jnp.ones((K, CL), jnp.float32)

    h = _mlp(x, W1, b1.reshape(1, -1), W2, b2.reshape(1, -1))
    gsum, gcnt, esum, ecnt = _scA(h, src, dst, hg_vertex, hg_edge, z128, z16, ones)
    e_feat = _ecomb(esum, ecnt)
    nsum, ncnt = _scB(e_feat, hg_edge, hg_vertex, z128, z16, ones)
    return _final(wv, h, gsum, gcnt, nsum, ncnt)


# speedup vs baseline: 4.2657x; 4.2657x over previous
import functools

import jax
import jax.numpy as jnp
from jax import lax
from jax.experimental import pallas as pl
from jax.experimental.pallas import tpu as pltpu
from jax.experimental.pallas import tpu_sc as plsc

N = 10000
C = 128
E = 320000
NNZ = 320000
NHE = 2000
K = 80
NSUB = 16
CL = 16

_mesh = lambda: plsc.VectorSubcoreMesh(core_axis_name="c", subcore_axis_name="s")


def _mlp_body(x_ref, w1_ref, b1_ref, w2_ref, b2_ref, o_ref):
    z = jnp.dot(x_ref[...], w1_ref[...], preferred_element_type=jnp.float32)
    z = z + b1_ref[...]
    z = jnp.where(z >= 0.0, z, 0.2 * z)
    o_ref[...] = jnp.dot(z, w2_ref[...], preferred_element_type=jnp.float32) + b2_ref[...]


def _mlp(x, W1, b1, W2, b2):
    R = 1000
    return pl.pallas_call(
        _mlp_body,
        grid=(N // R,),
        in_specs=[
            pl.BlockSpec((R, C), lambda i: (i, 0)),
            pl.BlockSpec((C, C // 2), lambda i: (0, 0)),
            pl.BlockSpec((1, C // 2), lambda i: (0, 0)),
            pl.BlockSpec((C // 2, C), lambda i: (0, 0)),
            pl.BlockSpec((1, C), lambda i: (0, 0)),
        ],
        out_specs=pl.BlockSpec((R, C), lambda i: (i, 0)),
        out_shape=jax.ShapeDtypeStruct((N, C), jnp.float32),
    )(x, W1, b1, W2, b2)


def _sc_pass(s, table_hbm, gidx_hbm, sidx_hbm, z128_hbm, z16_hbm, ones_hbm,
             sum_out, cnt_out, gidx_v, sidx_v, rows_v, ones_v, acc_sh, cnt_sh,
             rows_per_tile, base0, nchunks):
    pltpu.sync_copy(z128_hbm.at[pl.ds(0, rows_per_tile)],
                    acc_sh.at[pl.ds(s * rows_per_tile, rows_per_tile)])
    pltpu.sync_copy(z16_hbm.at[pl.ds(0, rows_per_tile)],
                    cnt_sh.at[pl.ds(s * rows_per_tile, rows_per_tile)])
    pltpu.sync_copy(ones_hbm, ones_v)
    plsc.subcore_barrier()

    def chunk(i, carry):
        base = base0 + i * K
        pltpu.sync_copy(gidx_hbm.at[pl.ds(base, K)], gidx_v)
        pltpu.sync_copy(sidx_hbm.at[pl.ds(base, K)], sidx_v)
        pltpu.sync_copy(table_hbm.at[gidx_v], rows_v)
        pltpu.sync_copy(rows_v, acc_sh.at[sidx_v], add=True)
        pltpu.sync_copy(ones_v, cnt_sh.at[sidx_v], add=True)
        return carry

    lax.fori_loop(0, nchunks, chunk, 0)
    plsc.subcore_barrier()
    pltpu.sync_copy(acc_sh.at[pl.ds(s * rows_per_tile, rows_per_tile)],
                    sum_out.at[pl.ds(s * rows_per_tile, rows_per_tile)])
    pltpu.sync_copy(cnt_sh.at[pl.ds(s * rows_per_tile, rows_per_tile)],
                    cnt_out.at[pl.ds(s * rows_per_tile, rows_per_tile)])


def _scA_body(h_hbm, src_hbm, dst_hbm, hgv_hbm, hge_hbm, z128_hbm, z16_hbm,
              ones_hbm, gsum_out, gcnt_out, esum_out, ecnt_out,
              gidx_v, sidx_v, rows_v, ones_v, acc_sh, cnt_sh):
    c = lax.axis_index("c")
    s = lax.axis_index("s")
    per_tile = E // NSUB

    @pl.when(c == 0)
    def _():
        _sc_pass(s, h_hbm, src_hbm, dst_hbm, z128_hbm, z16_hbm, ones_hbm,
                 gsum_out, gcnt_out, gidx_v, sidx_v, rows_v, ones_v,
                 acc_sh, cnt_sh, N // NSUB, s * per_tile, per_tile // K)

    @pl.when(c == 1)
    def _():
        _sc_pass(s, h_hbm, hgv_hbm, hge_hbm, z128_hbm, z16_hbm, ones_hbm,
                 esum_out, ecnt_out, gidx_v, sidx_v, rows_v, ones_v,
                 acc_sh, cnt_sh, NHE // NSUB, s * per_tile, per_tile // K)


def _scA(h, src, dst, hgv, hge, z128, z16, ones):
    f = pl.kernel(
        _scA_body,
        out_type=[
            jax.ShapeDtypeStruct((N, C), jnp.float32),
            jax.ShapeDtypeStruct((N, CL), jnp.float32),
            jax.ShapeDtypeStruct((NHE, C), jnp.float32),
            jax.ShapeDtypeStruct((NHE, CL), jnp.float32),
        ],
        mesh=_mesh(),
        compiler_params=pltpu.CompilerParams(use_tc_tiling_on_sc=False),
        scratch_types=[
            pltpu.VMEM((K,), jnp.int32),
            pltpu.VMEM((K,), jnp.int32),
            pltpu.VMEM((K, C), jnp.float32),
            pltpu.VMEM((K, CL), jnp.float32),
            pltpu.VMEM_SHARED((N, C), jnp.float32),
            pltpu.VMEM_SHARED((N, CL), jnp.float32),
        ],
    )
    return f(h, src, dst, hgv, hge, z128, z16, ones)


def _scB_body(ef_hbm, hge_hbm, hgv_hbm, z128_hbm, z16_hbm, ones_hbm,
              nsum_out, ncnt_out, gidx_v, sidx_v, rows_v, ones_v, acc_sh, cnt_sh):
    c = lax.axis_index("c")
    s = lax.axis_index("s")
    wid = s * 2 + c
    per_w = NNZ // (2 * NSUB)
    _sc_pass(s, ef_hbm, hge_hbm, hgv_hbm, z128_hbm, z16_hbm, ones_hbm,
             nsum_out.at[c], ncnt_out.at[c], gidx_v, sidx_v, rows_v, ones_v,
             acc_sh, cnt_sh, N // NSUB, wid * per_w, per_w // K)


def _scB(ef, hge, hgv, z128, z16, ones):
    f = pl.kernel(
        _scB_body,
        out_type=[
            jax.ShapeDtypeStruct((2, N, C), jnp.float32),
            jax.ShapeDtypeStruct((2, N, CL), jnp.float32),
        ],
        mesh=_mesh(),
        compiler_params=pltpu.CompilerParams(use_tc_tiling_on_sc=False),
        scratch_types=[
            pltpu.VMEM((K,), jnp.int32),
            pltpu.VMEM((K,), jnp.int32),
            pltpu.VMEM((K, C), jnp.float32),
            pltpu.VMEM((K, CL), jnp.float32),
            pltpu.VMEM_SHARED((N, C), jnp.float32),
            pltpu.VMEM_SHARED((N, CL), jnp.float32),
        ],
    )
    return f(ef, hge, hgv, z128, z16, ones)


def _ecomb_body(es_ref, ec_ref, o_ref):
    cnt = jnp.maximum(ec_ref[:, 0:1], 1.0)
    o_ref[...] = es_ref[...] / cnt


def _ecomb(esum, ecnt):
    return pl.pallas_call(
        _ecomb_body,
        out_shape=jax.ShapeDtypeStruct((NHE, C), jnp.float32),
    )(esum, ecnt)


def _final_body(wv_ref, h_ref, gs_ref, gc_ref, ns_ref, nc_ref, o_ref):
    w1 = wv_ref[0, 0]
    w2 = wv_ref[0, 1]
    xg = gs_ref[...] / jnp.maximum(gc_ref[:, 0:1], 1.0)
    nsum = ns_ref[0] + ns_ref[1]
    ncnt = nc_ref[0, :, 0:1] + nc_ref[1, :, 0:1]
    xhg = nsum / jnp.maximum(ncnt, 1.0)
    out = w1 * ((xg + xhg) * 0.5) + w2 * h_ref[...]
    o_ref[...] = jnp.where(out >= 0.0, out, 0.2 * out)


def _final(wv, h, gsum, gcnt, nsum, ncnt):
    R = 1000
    return pl.pallas_call(
        _final_body,
        grid=(N // R,),
        in_specs=[
            pl.BlockSpec(memory_space=pltpu.SMEM),
            pl.BlockSpec((R, C), lambda i: (i, 0)),
            pl.BlockSpec((R, C), lambda i: (i, 0)),
            pl.BlockSpec((R, CL), lambda i: (i, 0)),
            pl.BlockSpec((2, R, C), lambda i: (0, i, 0)),
            pl.BlockSpec((2, R, CL), lambda i: (0, i, 0)),
        ],
        out_specs=pl.BlockSpec((R, C), lambda i: (i, 0)),
        out_shape=jax.ShapeDtypeStruct((N, C), jnp.float32),
    )(wv, h, gsum, gcnt, nsum, ncnt)


def kernel(x, w, W1, b1, W2, b2, graph_edge_index, hg_vertex, hg_edge):
    src = graph_edge_index[0]
    dst = graph_edge_index[1]
    ew = jnp.exp(w)
    wv = (ew / jnp.sum(ew)).reshape(1, 2)
    z128 = jnp.zeros((N // NSUB, C), jnp.float32)
    z16 = jnp.zeros((N // NSUB, CL), jnp.float32)
    ones = jnp.ones((K, CL), jnp.float32)

    h = _mlp(x, W1, b1.reshape(1, -1), W2, b2.reshape(1, -1))
    gsum, gcnt, esum, ecnt = _scA(h, src, dst, hg_vertex, hg_edge, z128, z16, ones)
    e_feat = _ecomb(esum, ecnt)
    nsum, ncnt = _scB(e_feat, hg_edge, hg_vertex, z128, z16, ones)
    return _final(wv, h, gsum, gcnt, nsum, ncnt)


# speedup vs baseline: 7.7366x; 1.8137x over previous
import functools

import jax
import jax.numpy as jnp
from jax import lax
from jax.experimental import pallas as pl
from jax.experimental.pallas import tpu as pltpu
from jax.experimental.pallas import tpu_sc as plsc

N = 10000
C = 128
E = 320000
NNZ = 320000
NHE = 2000
K = 80
NSUB = 16
CL = 16

_mesh = lambda: plsc.VectorSubcoreMesh(core_axis_name="c", subcore_axis_name="s")


def _mlp_body(x_ref, w1_ref, b1_ref, w2_ref, b2_ref, o_ref):
    z = jnp.dot(x_ref[...], w1_ref[...], preferred_element_type=jnp.float32)
    z = z + b1_ref[...]
    z = jnp.where(z >= 0.0, z, 0.2 * z)
    o_ref[...] = jnp.dot(z, w2_ref[...], preferred_element_type=jnp.float32) + b2_ref[...]


def _mlp(x, W1, b1, W2, b2):
    R = 1000
    return pl.pallas_call(
        _mlp_body,
        grid=(N // R,),
        in_specs=[
            pl.BlockSpec((R, C), lambda i: (i, 0)),
            pl.BlockSpec((C, C // 2), lambda i: (0, 0)),
            pl.BlockSpec((1, C // 2), lambda i: (0, 0)),
            pl.BlockSpec((C // 2, C), lambda i: (0, 0)),
            pl.BlockSpec((1, C), lambda i: (0, 0)),
        ],
        out_specs=pl.BlockSpec((R, C), lambda i: (i, 0)),
        out_shape=jax.ShapeDtypeStruct((N, C), jnp.float32),
    )(x, W1, b1, W2, b2)


def _sc_pass(s, table_hbm, idx2_hbm, z128_hbm, z16_hbm, ones_hbm,
             sum_out, cnt_out, idxA, idxB, rowsA, rowsB, ones_v,
             acc_sh, cnt_sh, semA, semB, semIA, semIB,
             rows_per_tile, chunk0, nchunks):
    pltpu.sync_copy(z128_hbm.at[pl.ds(0, rows_per_tile)],
                    acc_sh.at[pl.ds(s * rows_per_tile, rows_per_tile)])
    pltpu.sync_copy(z16_hbm.at[pl.ds(0, rows_per_tile)],
                    cnt_sh.at[pl.ds(s * rows_per_tile, rows_per_tile)])
    pltpu.sync_copy(ones_hbm, ones_v)
    pltpu.sync_copy(idx2_hbm.at[chunk0], idxA)
    pltpu.async_copy(table_hbm.at[idxA.at[0]], rowsA, semA)
    pltpu.async_copy(idx2_hbm.at[chunk0 + 1], idxB, semIB)
    plsc.subcore_barrier()

    def scatter(rows_v, idx_v):
        pltpu.sync_copy(rows_v, acc_sh.at[idx_v.at[1]], add=True)
        pltpu.sync_copy(ones_v, cnt_sh.at[idx_v.at[1]], add=True)

    def body(j, carry):
        i0 = 2 * j
        i1 = i0 + 1
        pltpu.make_async_copy(idx2_hbm.at[chunk0 + i1], idxB, semIB).wait()
        pltpu.async_copy(table_hbm.at[idxB.at[0]], rowsB, semB)
        pltpu.make_async_copy(table_hbm.at[idxA.at[0]], rowsA, semA).wait()
        scatter(rowsA, idxA)

        @pl.when(i0 + 2 < nchunks)
        def _():
            pltpu.async_copy(idx2_hbm.at[chunk0 + i0 + 2], idxA, semIA)
            pltpu.make_async_copy(idx2_hbm.at[chunk0 + i0 + 2], idxA, semIA).wait()
            pltpu.async_copy(table_hbm.at[idxA.at[0]], rowsA, semA)

        pltpu.make_async_copy(table_hbm.at[idxB.at[0]], rowsB, semB).wait()
        scatter(rowsB, idxB)

        @pl.when(i1 + 2 < nchunks)
        def _():
            pltpu.async_copy(idx2_hbm.at[chunk0 + i1 + 2], idxB, semIB)

        return carry

    lax.fori_loop(0, nchunks // 2, body, 0)
    if nchunks % 2:
        pltpu.make_async_copy(table_hbm.at[idxA.at[0]], rowsA, semA).wait()
        scatter(rowsA, idxA)
    plsc.subcore_barrier()
    pltpu.sync_copy(acc_sh.at[pl.ds(s * rows_per_tile, rows_per_tile)],
                    sum_out.at[pl.ds(s * rows_per_tile, rows_per_tile)])
    pltpu.sync_copy(cnt_sh.at[pl.ds(s * rows_per_tile, rows_per_tile)],
                    cnt_out.at[pl.ds(s * rows_per_tile, rows_per_tile)])


def _scA_body(h_hbm, gidx2_hbm, hidx2_hbm, z128_hbm, z16_hbm,
              ones_hbm, gsum_out, gcnt_out, esum_out, ecnt_out,
              idxA, idxB, rowsA, rowsB, ones_v, acc_sh, cnt_sh,
              semA, semB, semIA, semIB):
    c = lax.axis_index("c")
    s = lax.axis_index("s")
    nch = E // NSUB // K

    @pl.when(c == 0)
    def _():
        _sc_pass(s, h_hbm, gidx2_hbm, z128_hbm, z16_hbm, ones_hbm,
                 gsum_out, gcnt_out, idxA, idxB, rowsA, rowsB, ones_v,
                 acc_sh, cnt_sh, semA, semB, semIA, semIB,
                 N // NSUB, s * nch, nch)

    @pl.when(c == 1)
    def _():
        _sc_pass(s, h_hbm, hidx2_hbm, z128_hbm, z16_hbm, ones_hbm,
                 esum_out, ecnt_out, idxA, idxB, rowsA, rowsB, ones_v,
                 acc_sh, cnt_sh, semA, semB, semIA, semIB,
                 NHE // NSUB, s * nch, nch)


def _scA(h, gidx2, hidx2, z128, z16, ones):
    f = pl.kernel(
        _scA_body,
        out_type=[
            jax.ShapeDtypeStruct((N, C), jnp.float32),
            jax.ShapeDtypeStruct((N, CL), jnp.float32),
            jax.ShapeDtypeStruct((NHE, C), jnp.float32),
            jax.ShapeDtypeStruct((NHE, CL), jnp.float32),
        ],
        mesh=_mesh(),
        compiler_params=pltpu.CompilerParams(use_tc_tiling_on_sc=False),
        scratch_types=[
            pltpu.VMEM((2, K), jnp.int32),
            pltpu.VMEM((2, K), jnp.int32),
            pltpu.VMEM((K, C), jnp.float32),
            pltpu.VMEM((K, C), jnp.float32),
            pltpu.VMEM((K, CL), jnp.float32),
            pltpu.VMEM_SHARED((N, C), jnp.float32),
            pltpu.VMEM_SHARED((N, CL), jnp.float32),
            pltpu.SemaphoreType.DMA,
            pltpu.SemaphoreType.DMA,
            pltpu.SemaphoreType.DMA,
            pltpu.SemaphoreType.DMA,
        ],
    )
    return f(h, gidx2, hidx2, z128, z16, ones)


def _scB_body(ef_hbm, idx2_hbm, z128_hbm, z16_hbm, ones_hbm,
              nsum_out, ncnt_out, idxA, idxB, rowsA, rowsB, ones_v,
              acc_sh, cnt_sh, semA, semB, semIA, semIB):
    c = lax.axis_index("c")
    s = lax.axis_index("s")
    wid = s * 2 + c
    nch = NNZ // (2 * NSUB) // K
    _sc_pass(s, ef_hbm, idx2_hbm, z128_hbm, z16_hbm, ones_hbm,
             nsum_out.at[c], ncnt_out.at[c], idxA, idxB, rowsA, rowsB,
             ones_v, acc_sh, cnt_sh, semA, semB, semIA, semIB,
             N // NSUB, wid * nch, nch)


def _scB(ef, idx2, z128, z16, ones):
    f = pl.kernel(
        _scB_body,
        out_type=[
            jax.ShapeDtypeStruct((2, N, C), jnp.float32),
            jax.ShapeDtypeStruct((2, N, CL), jnp.float32),
        ],
        mesh=_mesh(),
        compiler_params=pltpu.CompilerParams(use_tc_tiling_on_sc=False),
        scratch_types=[
            pltpu.VMEM((2, K), jnp.int32),
            pltpu.VMEM((2, K), jnp.int32),
            pltpu.VMEM((K, C), jnp.float32),
            pltpu.VMEM((K, C), jnp.float32),
            pltpu.VMEM((K, CL), jnp.float32),
            pltpu.VMEM_SHARED((N, C), jnp.float32),
            pltpu.VMEM_SHARED((N, CL), jnp.float32),
            pltpu.SemaphoreType.DMA,
            pltpu.SemaphoreType.DMA,
            pltpu.SemaphoreType.DMA,
            pltpu.SemaphoreType.DMA,
        ],
    )
    return f(ef, idx2, z128, z16, ones)


def _ecomb_body(es_ref, ec_ref, o_ref):
    cnt = jnp.maximum(ec_ref[:, 0:1], 1.0)
    o_ref[...] = es_ref[...] / cnt


def _ecomb(esum, ecnt):
    return pl.pallas_call(
        _ecomb_body,
        out_shape=jax.ShapeDtypeStruct((NHE, C), jnp.float32),
    )(esum, ecnt)


def _final_body(wv_ref, h_ref, gs_ref, gc_ref, ns_ref, nc_ref, o_ref):
    w1 = wv_ref[0, 0]
    w2 = wv_ref[0, 1]
    xg = gs_ref[...] / jnp.maximum(gc_ref[:, 0:1], 1.0)
    nsum = ns_ref[0] + ns_ref[1]
    ncnt = nc_ref[0, :, 0:1] + nc_ref[1, :, 0:1]
    xhg = nsum / jnp.maximum(ncnt, 1.0)
    out = w1 * ((xg + xhg) * 0.5) + w2 * h_ref[...]
    o_ref[...] = jnp.where(out >= 0.0, out, 0.2 * out)


def _final(wv, h, gsum, gcnt, nsum, ncnt):
    R = 1000
    return pl.pallas_call(
        _final_body,
        grid=(N // R,),
        in_specs=[
            pl.BlockSpec(memory_space=pltpu.SMEM),
            pl.BlockSpec((R, C), lambda i: (i, 0)),
            pl.BlockSpec((R, C), lambda i: (i, 0)),
            pl.BlockSpec((R, CL), lambda i: (i, 0)),
            pl.BlockSpec((2, R, C), lambda i: (0, i, 0)),
            pl.BlockSpec((2, R, CL), lambda i: (0, i, 0)),
        ],
        out_specs=pl.BlockSpec((R, C), lambda i: (i, 0)),
        out_shape=jax.ShapeDtypeStruct((N, C), jnp.float32),
    )(wv, h, gsum, gcnt, nsum, ncnt)


def kernel(x, w, W1, b1, W2, b2, graph_edge_index, hg_vertex, hg_edge):
    src = graph_edge_index[0]
    dst = graph_edge_index[1]
    ew = jnp.exp(w)
    wv = (ew / jnp.sum(ew)).reshape(1, 2)
    z128 = jnp.zeros((N // NSUB, C), jnp.float32)
    z16 = jnp.zeros((N // NSUB, CL), jnp.float32)
    ones = jnp.ones((K, CL), jnp.float32)

    gidx2 = jnp.stack([src.reshape(-1, K), dst.reshape(-1, K)], axis=1)
    hidx2 = jnp.stack([hg_vertex.reshape(-1, K), hg_edge.reshape(-1, K)], axis=1)
    eidx2 = jnp.stack([hg_edge.reshape(-1, K), hg_vertex.reshape(-1, K)], axis=1)

    h = _mlp(x, W1, b1.reshape(1, -1), W2, b2.reshape(1, -1))
    gsum, gcnt, esum, ecnt = _scA(h, gidx2, hidx2, z128, z16, ones)
    e_feat = _ecomb(esum, ecnt)
    nsum, ncnt = _scB(e_feat, eidx2, z128, z16, ones)
    return _final(wv, h, gsum, gcnt, nsum, ncnt)


# speedup vs baseline: 8.2578x; 1.0674x over previous
import functools

import jax
import jax.numpy as jnp
from jax import lax
from jax.experimental import pallas as pl
from jax.experimental.pallas import tpu as pltpu
from jax.experimental.pallas import tpu_sc as plsc

N = 10000
C = 128
E = 320000
NNZ = 320000
NHE = 2000
K = 100
NSUB = 16
CL = 16

_mesh = lambda: plsc.VectorSubcoreMesh(core_axis_name="c", subcore_axis_name="s")


def _mlp_body(x_ref, w1_ref, b1_ref, w2_ref, b2_ref, o_ref):
    z = jnp.dot(x_ref[...], w1_ref[...], preferred_element_type=jnp.float32)
    z = z + b1_ref[...]
    z = jnp.where(z >= 0.0, z, 0.2 * z)
    o_ref[...] = jnp.dot(z, w2_ref[...], preferred_element_type=jnp.float32) + b2_ref[...]


def _mlp(x, W1, b1, W2, b2):
    R = 1000
    return pl.pallas_call(
        _mlp_body,
        grid=(N // R,),
        in_specs=[
            pl.BlockSpec((R, C), lambda i: (i, 0)),
            pl.BlockSpec((C, C // 2), lambda i: (0, 0)),
            pl.BlockSpec((1, C // 2), lambda i: (0, 0)),
            pl.BlockSpec((C // 2, C), lambda i: (0, 0)),
            pl.BlockSpec((1, C), lambda i: (0, 0)),
        ],
        out_specs=pl.BlockSpec((R, C), lambda i: (i, 0)),
        out_shape=jax.ShapeDtypeStruct((N, C), jnp.float32),
    )(x, W1, b1, W2, b2)


def _sc_pass(s, table_hbm, idx2_hbm, z128_hbm, z16_hbm, ones_hbm,
             sum_out, cnt_out, idxA, idxB, rowsA, rowsB, ones_v,
             acc_sh, cnt_sh, semA, semB, semIA, semIB,
             rows_per_tile, chunk0, nchunks):
    pltpu.sync_copy(z128_hbm.at[pl.ds(0, rows_per_tile)],
                    acc_sh.at[pl.ds(s * rows_per_tile, rows_per_tile)])
    pltpu.sync_copy(z16_hbm.at[pl.ds(0, rows_per_tile)],
                    cnt_sh.at[pl.ds(s * rows_per_tile, rows_per_tile)])
    pltpu.sync_copy(ones_hbm, ones_v)
    pltpu.sync_copy(idx2_hbm.at[chunk0], idxA)
    pltpu.async_copy(table_hbm.at[idxA.at[0]], rowsA, semA)
    pltpu.async_copy(idx2_hbm.at[chunk0 + 1], idxB, semIB)
    plsc.subcore_barrier()

    def scatter(rows_v, idx_v):
        pltpu.sync_copy(rows_v, acc_sh.at[idx_v.at[1]], add=True)
        pltpu.sync_copy(ones_v, cnt_sh.at[idx_v.at[1]], add=True)

    def body(j, carry):
        i0 = 2 * j
        i1 = i0 + 1
        pltpu.make_async_copy(idx2_hbm.at[chunk0 + i1], idxB, semIB).wait()
        pltpu.async_copy(table_hbm.at[idxB.at[0]], rowsB, semB)
        pltpu.make_async_copy(table_hbm.at[idxA.at[0]], rowsA, semA).wait()
        scatter(rowsA, idxA)

        @pl.when(i0 + 2 < nchunks)
        def _():
            pltpu.async_copy(idx2_hbm.at[chunk0 + i0 + 2], idxA, semIA)
            pltpu.make_async_copy(idx2_hbm.at[chunk0 + i0 + 2], idxA, semIA).wait()
            pltpu.async_copy(table_hbm.at[idxA.at[0]], rowsA, semA)

        pltpu.make_async_copy(table_hbm.at[idxB.at[0]], rowsB, semB).wait()
        scatter(rowsB, idxB)

        @pl.when(i1 + 2 < nchunks)
        def _():
            pltpu.async_copy(idx2_hbm.at[chunk0 + i1 + 2], idxB, semIB)

        return carry

    lax.fori_loop(0, nchunks // 2, body, 0)
    if nchunks % 2:
        pltpu.make_async_copy(table_hbm.at[idxA.at[0]], rowsA, semA).wait()
        scatter(rowsA, idxA)
    plsc.subcore_barrier()
    pltpu.sync_copy(acc_sh.at[pl.ds(s * rows_per_tile, rows_per_tile)],
                    sum_out.at[pl.ds(s * rows_per_tile, rows_per_tile)])
    pltpu.sync_copy(cnt_sh.at[pl.ds(s * rows_per_tile, rows_per_tile)],
                    cnt_out.at[pl.ds(s * rows_per_tile, rows_per_tile)])


def _scA_body(h_hbm, gidx2_hbm, hidx2_hbm, z128_hbm, z16_hbm,
              ones_hbm, gsum_out, gcnt_out, esum_out, ecnt_out,
              idxA, idxB, rowsA, rowsB, ones_v, acc_sh, cnt_sh,
              semA, semB, semIA, semIB):
    c = lax.axis_index("c")
    s = lax.axis_index("s")
    nch = E // NSUB // K

    @pl.when(c == 0)
    def _():
        _sc_pass(s, h_hbm, gidx2_hbm, z128_hbm, z16_hbm, ones_hbm,
                 gsum_out, gcnt_out, idxA, idxB, rowsA, rowsB, ones_v,
                 acc_sh, cnt_sh, semA, semB, semIA, semIB,
                 N // NSUB, s * nch, nch)

    @pl.when(c == 1)
    def _():
        _sc_pass(s, h_hbm, hidx2_hbm, z128_hbm, z16_hbm, ones_hbm,
                 esum_out, ecnt_out, idxA, idxB, rowsA, rowsB, ones_v,
                 acc_sh, cnt_sh, semA, semB, semIA, semIB,
                 NHE // NSUB, s * nch, nch)


def _scA(h, gidx2, hidx2, z128, z16, ones):
    f = pl.kernel(
        _scA_body,
        out_type=[
            jax.ShapeDtypeStruct((N, C), jnp.float32),
            jax.ShapeDtypeStruct((N, CL), jnp.float32),
            jax.ShapeDtypeStruct((NHE, C), jnp.float32),
            jax.ShapeDtypeStruct((NHE, CL), jnp.float32),
        ],
        mesh=_mesh(),
        compiler_params=pltpu.CompilerParams(use_tc_tiling_on_sc=False),
        scratch_types=[
            pltpu.VMEM((2, K), jnp.int32),
            pltpu.VMEM((2, K), jnp.int32),
            pltpu.VMEM((K, C), jnp.float32),
            pltpu.VMEM((K, C), jnp.float32),
            pltpu.VMEM((K, CL), jnp.float32),
            pltpu.VMEM_SHARED((N, C), jnp.float32),
            pltpu.VMEM_SHARED((N, CL), jnp.float32),
            pltpu.SemaphoreType.DMA,
            pltpu.SemaphoreType.DMA,
            pltpu.SemaphoreType.DMA,
            pltpu.SemaphoreType.DMA,
        ],
    )
    return f(h, gidx2, hidx2, z128, z16, ones)


def _scB_body(ef_hbm, idx2_hbm, z128_hbm, z16_hbm, ones_hbm,
              nsum_out, ncnt_out, idxA, idxB, rowsA, rowsB, ones_v,
              acc_sh, cnt_sh, semA, semB, semIA, semIB):
    c = lax.axis_index("c")
    s = lax.axis_index("s")
    wid = s * 2 + c
    nch = NNZ // (2 * NSUB) // K
    _sc_pass(s, ef_hbm, idx2_hbm, z128_hbm, z16_hbm, ones_hbm,
             nsum_out.at[c], ncnt_out.at[c], idxA, idxB, rowsA, rowsB,
             ones_v, acc_sh, cnt_sh, semA, semB, semIA, semIB,
             N // NSUB, wid * nch, nch)


def _scB(ef, idx2, z128, z16, ones):
    f = pl.kernel(
        _scB_body,
        out_type=[
            jax.ShapeDtypeStruct((2, N, C), jnp.float32),
            jax.ShapeDtypeStruct((2, N, CL), jnp.float32),
        ],
        mesh=_mesh(),
        compiler_params=pltpu.CompilerParams(use_tc_tiling_on_sc=False),
        scratch_types=[
            pltpu.VMEM((2, K), jnp.int32),
            pltpu.VMEM((2, K), jnp.int32),
            pltpu.VMEM((K, C), jnp.float32),
            pltpu.VMEM((K, C), jnp.float32),
            pltpu.VMEM((K, CL), jnp.float32),
            pltpu.VMEM_SHARED((N, C), jnp.float32),
            pltpu.VMEM_SHARED((N, CL), jnp.float32),
            pltpu.SemaphoreType.DMA,
            pltpu.SemaphoreType.DMA,
            pltpu.SemaphoreType.DMA,
            pltpu.SemaphoreType.DMA,
        ],
    )
    return f(ef, idx2, z128, z16, ones)


def _ecomb_body(es_ref, ec_ref, o_ref):
    cnt = jnp.maximum(ec_ref[:, 0:1], 1.0)
    o_ref[...] = es_ref[...] / cnt


def _ecomb(esum, ecnt):
    return pl.pallas_call(
        _ecomb_body,
        out_shape=jax.ShapeDtypeStruct((NHE, C), jnp.float32),
    )(esum, ecnt)


def _final_body(wv_ref, h_ref, gs_ref, gc_ref, ns_ref, nc_ref, o_ref):
    w1 = wv_ref[0, 0]
    w2 = wv_ref[0, 1]
    xg = gs_ref[...] / jnp.maximum(gc_ref[:, 0:1], 1.0)
    nsum = ns_ref[0] + ns_ref[1]
    ncnt = nc_ref[0, :, 0:1] + nc_ref[1, :, 0:1]
    xhg = nsum / jnp.maximum(ncnt, 1.0)
    out = w1 * ((xg + xhg) * 0.5) + w2 * h_ref[...]
    o_ref[...] = jnp.where(out >= 0.0, out, 0.2 * out)


def _final(wv, h, gsum, gcnt, nsum, ncnt):
    R = 1000
    return pl.pallas_call(
        _final_body,
        grid=(N // R,),
        in_specs=[
            pl.BlockSpec(memory_space=pltpu.SMEM),
            pl.BlockSpec((R, C), lambda i: (i, 0)),
            pl.BlockSpec((R, C), lambda i: (i, 0)),
            pl.BlockSpec((R, CL), lambda i: (i, 0)),
            pl.BlockSpec((2, R, C), lambda i: (0, i, 0)),
            pl.BlockSpec((2, R, CL), lambda i: (0, i, 0)),
        ],
        out_specs=pl.BlockSpec((R, C), lambda i: (i, 0)),
        out_shape=jax.ShapeDtypeStruct((N, C), jnp.float32),
    )(wv, h, gsum, gcnt, nsum, ncnt)


def kernel(x, w, W1, b1, W2, b2, graph_edge_index, hg_vertex, hg_edge):
    src = graph_edge_index[0]
    dst = graph_edge_index[1]
    ew = jnp.exp(w)
    wv = (ew / jnp.sum(ew)).reshape(1, 2)
    z128 = jnp.zeros((N // NSUB, C), jnp.float32)
    z16 = jnp.zeros((N // NSUB, CL), jnp.float32)
    ones = jnp.ones((K, CL), jnp.float32)

    gidx2 = jnp.stack([src.reshape(-1, K), dst.reshape(-1, K)], axis=1)
    hidx2 = jnp.stack([hg_vertex.reshape(-1, K), hg_edge.reshape(-1, K)], axis=1)
    eidx2 = jnp.stack([hg_edge.reshape(-1, K), hg_vertex.reshape(-1, K)], axis=1)

    h = _mlp(x, W1, b1.reshape(1, -1), W2, b2.reshape(1, -1))
    gsum, gcnt, esum, ecnt = _scA(h, gidx2, hidx2, z128, z16, ones)
    e_feat = _ecomb(esum, ecnt)
    nsum, ncnt = _scB(e_feat, eidx2, z128, z16, ones)
    return _final(wv, h, gsum, gcnt, nsum, ncnt)


# speedup vs baseline: 8.3947x; 1.0166x over previous
import functools

import jax
import jax.numpy as jnp
from jax import lax
from jax.experimental import pallas as pl
from jax.experimental.pallas import tpu as pltpu
from jax.experimental.pallas import tpu_sc as plsc

N = 10000
C = 128
E = 320000
NNZ = 320000
NHE = 2000
K = 100
NSUB = 16
CL = 16

_mesh = lambda: plsc.VectorSubcoreMesh(core_axis_name="c", subcore_axis_name="s")


def _mlp_body(x_ref, w1_ref, b1_ref, w2_ref, b2_ref, o_ref):
    z = jnp.dot(x_ref[...], w1_ref[...], preferred_element_type=jnp.float32)
    z = z + b1_ref[...]
    z = jnp.where(z >= 0.0, z, 0.2 * z)
    o_ref[...] = jnp.dot(z, w2_ref[...], preferred_element_type=jnp.float32) + b2_ref[...]


def _mlp(x, W1, b1, W2, b2):
    R = 1000
    return pl.pallas_call(
        _mlp_body,
        grid=(N // R,),
        in_specs=[
            pl.BlockSpec((R, C), lambda i: (i, 0)),
            pl.BlockSpec((C, C // 2), lambda i: (0, 0)),
            pl.BlockSpec((1, C // 2), lambda i: (0, 0)),
            pl.BlockSpec((C // 2, C), lambda i: (0, 0)),
            pl.BlockSpec((1, C), lambda i: (0, 0)),
        ],
        out_specs=pl.BlockSpec((R, C), lambda i: (i, 0)),
        out_shape=jax.ShapeDtypeStruct((N, C), jnp.float32),
    )(x, W1, b1, W2, b2)


def _sc_pass(s, table_hbm, idx2_hbm, z128_hbm, z16_hbm, ones_hbm,
             sum_out, cnt_out, idxA, idxB, rowsA, rowsB, ones_v,
             acc_sh, cnt_sh, semA, semB, semIA, semIB, semO,
             rows_per_tile, chunk0, nchunks):
    pltpu.sync_copy(z128_hbm.at[pl.ds(0, rows_per_tile)],
                    acc_sh.at[pl.ds(s * rows_per_tile, rows_per_tile)])
    pltpu.sync_copy(z16_hbm.at[pl.ds(0, rows_per_tile)],
                    cnt_sh.at[pl.ds(s * rows_per_tile, rows_per_tile)])
    pltpu.sync_copy(ones_hbm, ones_v)
    pltpu.sync_copy(idx2_hbm.at[chunk0], idxA)
    pltpu.async_copy(table_hbm.at[idxA.at[0]], rowsA, semA)
    pltpu.async_copy(idx2_hbm.at[chunk0 + 1], idxB, semIB)
    plsc.subcore_barrier()

    def scatter(rows_v, idx_v):
        pltpu.async_copy(ones_v, cnt_sh.at[idx_v.at[1]], semO, add=True)
        pltpu.sync_copy(rows_v, acc_sh.at[idx_v.at[1]], add=True)
        pltpu.make_async_copy(ones_v, cnt_sh.at[idx_v.at[1]], semO).wait()

    def body(j, carry):
        i0 = 2 * j
        i1 = i0 + 1
        pltpu.make_async_copy(idx2_hbm.at[chunk0 + i1], idxB, semIB).wait()
        pltpu.async_copy(table_hbm.at[idxB.at[0]], rowsB, semB)
        pltpu.make_async_copy(table_hbm.at[idxA.at[0]], rowsA, semA).wait()
        scatter(rowsA, idxA)

        @pl.when(i0 + 2 < nchunks)
        def _():
            pltpu.async_copy(idx2_hbm.at[chunk0 + i0 + 2], idxA, semIA)
            pltpu.make_async_copy(idx2_hbm.at[chunk0 + i0 + 2], idxA, semIA).wait()
            pltpu.async_copy(table_hbm.at[idxA.at[0]], rowsA, semA)

        pltpu.make_async_copy(table_hbm.at[idxB.at[0]], rowsB, semB).wait()
        scatter(rowsB, idxB)

        @pl.when(i1 + 2 < nchunks)
        def _():
            pltpu.async_copy(idx2_hbm.at[chunk0 + i1 + 2], idxB, semIB)

        return carry

    lax.fori_loop(0, nchunks // 2, body, 0)
    if nchunks % 2:
        pltpu.make_async_copy(table_hbm.at[idxA.at[0]], rowsA, semA).wait()
        scatter(rowsA, idxA)
    plsc.subcore_barrier()
    pltpu.sync_copy(acc_sh.at[pl.ds(s * rows_per_tile, rows_per_tile)],
                    sum_out.at[pl.ds(s * rows_per_tile, rows_per_tile)])
    pltpu.sync_copy(cnt_sh.at[pl.ds(s * rows_per_tile, rows_per_tile)],
                    cnt_out.at[pl.ds(s * rows_per_tile, rows_per_tile)])


def _scA_body(h_hbm, gidx2_hbm, hidx2_hbm, z128_hbm, z16_hbm,
              ones_hbm, gsum_out, gcnt_out, esum_out, ecnt_out,
              idxA, idxB, rowsA, rowsB, ones_v, acc_sh, cnt_sh,
              semA, semB, semIA, semIB, semO):
    c = lax.axis_index("c")
    s = lax.axis_index("s")
    nch = E // NSUB // K

    @pl.when(c == 0)
    def _():
        _sc_pass(s, h_hbm, gidx2_hbm, z128_hbm, z16_hbm, ones_hbm,
                 gsum_out, gcnt_out, idxA, idxB, rowsA, rowsB, ones_v,
                 acc_sh, cnt_sh, semA, semB, semIA, semIB, semO,
                 N // NSUB, s * nch, nch)

    @pl.when(c == 1)
    def _():
        _sc_pass(s, h_hbm, hidx2_hbm, z128_hbm, z16_hbm, ones_hbm,
                 esum_out, ecnt_out, idxA, idxB, rowsA, rowsB, ones_v,
                 acc_sh, cnt_sh, semA, semB, semIA, semIB, semO,
                 NHE // NSUB, s * nch, nch)


def _scA(h, gidx2, hidx2, z128, z16, ones):
    f = pl.kernel(
        _scA_body,
        out_type=[
            jax.ShapeDtypeStruct((N, C), jnp.float32),
            jax.ShapeDtypeStruct((N, CL), jnp.float32),
            jax.ShapeDtypeStruct((NHE, C), jnp.float32),
            jax.ShapeDtypeStruct((NHE, CL), jnp.float32),
        ],
        mesh=_mesh(),
        compiler_params=pltpu.CompilerParams(use_tc_tiling_on_sc=False),
        scratch_types=[
            pltpu.VMEM((2, K), jnp.int32),
            pltpu.VMEM((2, K), jnp.int32),
            pltpu.VMEM((K, C), jnp.float32),
            pltpu.VMEM((K, C), jnp.float32),
            pltpu.VMEM((K, CL), jnp.float32),
            pltpu.VMEM_SHARED((N, C), jnp.float32),
            pltpu.VMEM_SHARED((N, CL), jnp.float32),
            pltpu.SemaphoreType.DMA,
            pltpu.SemaphoreType.DMA,
            pltpu.SemaphoreType.DMA,
            pltpu.SemaphoreType.DMA,
            pltpu.SemaphoreType.DMA,
        ],
    )
    return f(h, gidx2, hidx2, z128, z16, ones)


def _scB_body(ef_hbm, idx2_hbm, z128_hbm, z16_hbm, ones_hbm,
              nsum_out, ncnt_out, idxA, idxB, rowsA, rowsB, ones_v,
              acc_sh, cnt_sh, semA, semB, semIA, semIB, semO):
    c = lax.axis_index("c")
    s = lax.axis_index("s")
    wid = s * 2 + c
    nch = NNZ // (2 * NSUB) // K
    _sc_pass(s, ef_hbm, idx2_hbm, z128_hbm, z16_hbm, ones_hbm,
             nsum_out.at[c], ncnt_out.at[c], idxA, idxB, rowsA, rowsB,
             ones_v, acc_sh, cnt_sh, semA, semB, semIA, semIB, semO,
             N // NSUB, wid * nch, nch)


def _scB(ef, idx2, z128, z16, ones):
    f = pl.kernel(
        _scB_body,
        out_type=[
            jax.ShapeDtypeStruct((2, N, C), jnp.float32),
            jax.ShapeDtypeStruct((2, N, CL), jnp.float32),
        ],
        mesh=_mesh(),
        compiler_params=pltpu.CompilerParams(use_tc_tiling_on_sc=False),
        scratch_types=[
            pltpu.VMEM((2, K), jnp.int32),
            pltpu.VMEM((2, K), jnp.int32),
            pltpu.VMEM((K, C), jnp.float32),
            pltpu.VMEM((K, C), jnp.float32),
            pltpu.VMEM((K, CL), jnp.float32),
            pltpu.VMEM_SHARED((N, C), jnp.float32),
            pltpu.VMEM_SHARED((N, CL), jnp.float32),
            pltpu.SemaphoreType.DMA,
            pltpu.SemaphoreType.DMA,
            pltpu.SemaphoreType.DMA,
            pltpu.SemaphoreType.DMA,
            pltpu.SemaphoreType.DMA,
        ],
    )
    return f(ef, idx2, z128, z16, ones)


def _ecomb_body(es_ref, ec_ref, o_ref):
    cnt = jnp.maximum(ec_ref[:, 0:1], 1.0)
    o_ref[...] = es_ref[...] / cnt


def _ecomb(esum, ecnt):
    return pl.pallas_call(
        _ecomb_body,
        out_shape=jax.ShapeDtypeStruct((NHE, C), jnp.float32),
    )(esum, ecnt)


def _final_body(wv_ref, h_ref, gs_ref, gc_ref, ns_ref, nc_ref, o_ref):
    w1 = wv_ref[0, 0]
    w2 = wv_ref[0, 1]
    xg = gs_ref[...] / jnp.maximum(gc_ref[:, 0:1], 1.0)
    nsum = ns_ref[0] + ns_ref[1]
    ncnt = nc_ref[0, :, 0:1] + nc_ref[1, :, 0:1]
    xhg = nsum / jnp.maximum(ncnt, 1.0)
    out = w1 * ((xg + xhg) * 0.5) + w2 * h_ref[...]
    o_ref[...] = jnp.where(out >= 0.0, out, 0.2 * out)


def _final(wv, h, gsum, gcnt, nsum, ncnt):
    R = 1000
    return pl.pallas_call(
        _final_body,
        grid=(N // R,),
        in_specs=[
            pl.BlockSpec(memory_space=pltpu.SMEM),
            pl.BlockSpec((R, C), lambda i: (i, 0)),
            pl.BlockSpec((R, C), lambda i: (i, 0)),
            pl.BlockSpec((R, CL), lambda i: (i, 0)),
            pl.BlockSpec((2, R, C), lambda i: (0, i, 0)),
            pl.BlockSpec((2, R, CL), lambda i: (0, i, 0)),
        ],
        out_specs=pl.BlockSpec((R, C), lambda i: (i, 0)),
        out_shape=jax.ShapeDtypeStruct((N, C), jnp.float32),
    )(wv, h, gsum, gcnt, nsum, ncnt)


def kernel(x, w, W1, b1, W2, b2, graph_edge_index, hg_vertex, hg_edge):
    src = graph_edge_index[0]
    dst = graph_edge_index[1]
    ew = jnp.exp(w)
    wv = (ew / jnp.sum(ew)).reshape(1, 2)
    z128 = jnp.zeros((N // NSUB, C), jnp.float32)
    z16 = jnp.zeros((N // NSUB, CL), jnp.float32)
    ones = jnp.ones((K, CL), jnp.float32)

    gidx2 = jnp.stack([src.reshape(-1, K), dst.reshape(-1, K)], axis=1)
    hidx2 = jnp.stack([hg_vertex.reshape(-1, K), hg_edge.reshape(-1, K)], axis=1)
    eidx2 = jnp.stack([hg_edge.reshape(-1, K), hg_vertex.reshape(-1, K)], axis=1)

    h = _mlp(x, W1, b1.reshape(1, -1), W2, b2.reshape(1, -1))
    gsum, gcnt, esum, ecnt = _scA(h, gidx2, hidx2, z128, z16, ones)
    e_feat = _ecomb(esum, ecnt)
    nsum, ncnt = _scB(e_feat, eidx2, z128, z16, ones)
    return _final(wv, h, gsum, gcnt, nsum, ncnt)


# speedup vs baseline: 9.6028x; 1.1439x over previous
import functools

import jax
import jax.numpy as jnp
from jax import lax
from jax.experimental import pallas as pl
from jax.experimental.pallas import tpu as pltpu
from jax.experimental.pallas import tpu_sc as plsc

N = 10000
C = 128
E = 320000
NNZ = 320000
NHE = 2000
K = 100
NSUB = 16
CL = 16

_mesh = lambda: plsc.VectorSubcoreMesh(core_axis_name="c", subcore_axis_name="s")


def _mlp_body(x_ref, w1_ref, b1_ref, w2_ref, b2_ref, o_ref):
    z = jnp.dot(x_ref[...], w1_ref[...], preferred_element_type=jnp.float32)
    z = z + b1_ref[...]
    z = jnp.where(z >= 0.0, z, 0.2 * z)
    o_ref[...] = jnp.dot(z, w2_ref[...], preferred_element_type=jnp.float32) + b2_ref[...]


def _mlp(x, W1, b1, W2, b2):
    R = 1000
    return pl.pallas_call(
        _mlp_body,
        grid=(N // R,),
        in_specs=[
            pl.BlockSpec((R, C), lambda i: (i, 0)),
            pl.BlockSpec((C, C // 2), lambda i: (0, 0)),
            pl.BlockSpec((1, C // 2), lambda i: (0, 0)),
            pl.BlockSpec((C // 2, C), lambda i: (0, 0)),
            pl.BlockSpec((1, C), lambda i: (0, 0)),
        ],
        out_specs=pl.BlockSpec((R, C), lambda i: (i, 0)),
        out_shape=jax.ShapeDtypeStruct((N, C), jnp.float32),
    )(x, W1, b1, W2, b2)


def _sc_pass(s, table_hbm, idx2_hbm, z128_hbm, z16_hbm, ones_hbm,
             sum_out, cnt_out, idx4, rowsA, rowsB, ones_v,
             acc_sh, cnt_sh, semA, semB, semI0, semI1, semI2, semI3, semO,
             rows_per_tile, chunk0, nchunks):
    pltpu.sync_copy(z128_hbm.at[pl.ds(0, rows_per_tile)],
                    acc_sh.at[pl.ds(s * rows_per_tile, rows_per_tile)])
    pltpu.sync_copy(z16_hbm.at[pl.ds(0, rows_per_tile)],
                    cnt_sh.at[pl.ds(s * rows_per_tile, rows_per_tile)])
    pltpu.sync_copy(ones_hbm, ones_v)
    pltpu.sync_copy(idx2_hbm.at[chunk0], idx4.at[0])
    pltpu.sync_copy(idx2_hbm.at[chunk0 + 1], idx4.at[1])
    semI = (semI0, semI1, semI2, semI3)
    pltpu.async_copy(idx2_hbm.at[chunk0 + 2], idx4.at[2], semI[2])
    pltpu.async_copy(idx2_hbm.at[chunk0 + 3], idx4.at[3], semI[3])
    pltpu.async_copy(table_hbm.at[idx4.at[0, 0]], rowsA, semA)
    pltpu.async_copy(table_hbm.at[idx4.at[1, 0]], rowsB, semB)
    plsc.subcore_barrier()

    def process(j, q, rows_v, semR):
        c = 4 * j + q
        qn = (q + 2) % 4
        pltpu.make_async_copy(table_hbm.at[idx4.at[q, 0]], rows_v, semR).wait()
        pltpu.async_copy(ones_v, cnt_sh.at[idx4.at[q, 1]], semO, add=True)
        pltpu.sync_copy(rows_v, acc_sh.at[idx4.at[q, 1]], add=True)
        pltpu.make_async_copy(ones_v, cnt_sh.at[idx4.at[q, 1]], semO).wait()

        @pl.when(c + 4 < nchunks)
        def _():
            pltpu.async_copy(idx2_hbm.at[chunk0 + c + 4], idx4.at[q], semI[q])

        @pl.when(c + 2 < nchunks)
        def _():
            pltpu.make_async_copy(idx2_hbm.at[chunk0 + c + 2], idx4.at[qn],
                                  semI[qn]).wait()
            pltpu.async_copy(table_hbm.at[idx4.at[qn, 0]], rows_v, semR)

    def body(j, carry):
        process(j, 0, rowsA, semA)
        process(j, 1, rowsB, semB)
        process(j, 2, rowsA, semA)
        process(j, 3, rowsB, semB)
        return carry

    lax.fori_loop(0, nchunks // 4, body, 0)
    plsc.subcore_barrier()
    pltpu.sync_copy(acc_sh.at[pl.ds(s * rows_per_tile, rows_per_tile)],
                    sum_out.at[pl.ds(s * rows_per_tile, rows_per_tile)])
    pltpu.sync_copy(cnt_sh.at[pl.ds(s * rows_per_tile, rows_per_tile)],
                    cnt_out.at[pl.ds(s * rows_per_tile, rows_per_tile)])


def _scA_body(h_hbm, gidx2_hbm, hidx2_hbm, z128_hbm, z16_hbm,
              ones_hbm, gsum_out, gcnt_out, esum_out, ecnt_out,
              idx4, rowsA, rowsB, ones_v, acc_sh, cnt_sh,
              semA, semB, semI0, semI1, semI2, semI3, semO):
    c = lax.axis_index("c")
    s = lax.axis_index("s")
    nch = E // NSUB // K

    @pl.when(c == 0)
    def _():
        _sc_pass(s, h_hbm, gidx2_hbm, z128_hbm, z16_hbm, ones_hbm,
                 gsum_out, gcnt_out, idx4, rowsA, rowsB, ones_v,
                 acc_sh, cnt_sh, semA, semB, semI0, semI1, semI2, semI3, semO,
                 N // NSUB, s * nch, nch)

    @pl.when(c == 1)
    def _():
        _sc_pass(s, h_hbm, hidx2_hbm, z128_hbm, z16_hbm, ones_hbm,
                 esum_out, ecnt_out, idx4, rowsA, rowsB, ones_v,
                 acc_sh, cnt_sh, semA, semB, semI0, semI1, semI2, semI3, semO,
                 NHE // NSUB, s * nch, nch)


def _scA(h, gidx2, hidx2, z128, z16, ones):
    f = pl.kernel(
        _scA_body,
        out_type=[
            jax.ShapeDtypeStruct((N, C), jnp.float32),
            jax.ShapeDtypeStruct((N, CL), jnp.float32),
            jax.ShapeDtypeStruct((NHE, C), jnp.float32),
            jax.ShapeDtypeStruct((NHE, CL), jnp.float32),
        ],
        mesh=_mesh(),
        compiler_params=pltpu.CompilerParams(use_tc_tiling_on_sc=False),
        scratch_types=[
            pltpu.VMEM((4, 2, K), jnp.int32),
            pltpu.VMEM((K, C), jnp.float32),
            pltpu.VMEM((K, C), jnp.float32),
            pltpu.VMEM((K, CL), jnp.float32),
            pltpu.VMEM_SHARED((N, C), jnp.float32),
            pltpu.VMEM_SHARED((N, CL), jnp.float32),
            pltpu.SemaphoreType.DMA,
            pltpu.SemaphoreType.DMA,
            pltpu.SemaphoreType.DMA,
            pltpu.SemaphoreType.DMA,
            pltpu.SemaphoreType.DMA,
            pltpu.SemaphoreType.DMA,
            pltpu.SemaphoreType.DMA,
        ],
    )
    return f(h, gidx2, hidx2, z128, z16, ones)


def _scB_body(ef_hbm, idx2_hbm, z128_hbm, z16_hbm, ones_hbm,
              nsum_out, ncnt_out, idx4, rowsA, rowsB, ones_v,
              acc_sh, cnt_sh, semA, semB, semI0, semI1, semI2, semI3, semO):
    c = lax.axis_index("c")
    s = lax.axis_index("s")
    wid = s * 2 + c
    nch = NNZ // (2 * NSUB) // K
    _sc_pass(s, ef_hbm, idx2_hbm, z128_hbm, z16_hbm, ones_hbm,
             nsum_out.at[c], ncnt_out.at[c], idx4, rowsA, rowsB,
             ones_v, acc_sh, cnt_sh, semA, semB, semI0, semI1, semI2, semI3, semO,
             N // NSUB, wid * nch, nch)


def _scB(ef, idx2, z128, z16, ones):
    f = pl.kernel(
        _scB_body,
        out_type=[
            jax.ShapeDtypeStruct((2, N, C), jnp.float32),
            jax.ShapeDtypeStruct((2, N, CL), jnp.float32),
        ],
        mesh=_mesh(),
        compiler_params=pltpu.CompilerParams(use_tc_tiling_on_sc=False),
        scratch_types=[
            pltpu.VMEM((4, 2, K), jnp.int32),
            pltpu.VMEM((K, C), jnp.float32),
            pltpu.VMEM((K, C), jnp.float32),
            pltpu.VMEM((K, CL), jnp.float32),
            pltpu.VMEM_SHARED((N, C), jnp.float32),
            pltpu.VMEM_SHARED((N, CL), jnp.float32),
            pltpu.SemaphoreType.DMA,
            pltpu.SemaphoreType.DMA,
            pltpu.SemaphoreType.DMA,
            pltpu.SemaphoreType.DMA,
            pltpu.SemaphoreType.DMA,
            pltpu.SemaphoreType.DMA,
            pltpu.SemaphoreType.DMA,
        ],
    )
    return f(ef, idx2, z128, z16, ones)


def _ecomb_body(es_ref, ec_ref, o_ref):
    cnt = jnp.maximum(ec_ref[:, 0:1], 1.0)
    o_ref[...] = es_ref[...] / cnt


def _ecomb(esum, ecnt):
    return pl.pallas_call(
        _ecomb_body,
        out_shape=jax.ShapeDtypeStruct((NHE, C), jnp.float32),
    )(esum, ecnt)


def _final_body(wv_ref, h_ref, gs_ref, gc_ref, ns_ref, nc_ref, o_ref):
    w1 = wv_ref[0, 0]
    w2 = wv_ref[0, 1]
    xg = gs_ref[...] / jnp.maximum(gc_ref[:, 0:1], 1.0)
    nsum = ns_ref[0] + ns_ref[1]
    ncnt = nc_ref[0, :, 0:1] + nc_ref[1, :, 0:1]
    xhg = nsum / jnp.maximum(ncnt, 1.0)
    out = w1 * ((xg + xhg) * 0.5) + w2 * h_ref[...]
    o_ref[...] = jnp.where(out >= 0.0, out, 0.2 * out)


def _final(wv, h, gsum, gcnt, nsum, ncnt):
    R = 1000
    return pl.pallas_call(
        _final_body,
        grid=(N // R,),
        in_specs=[
            pl.BlockSpec(memory_space=pltpu.SMEM),
            pl.BlockSpec((R, C), lambda i: (i, 0)),
            pl.BlockSpec((R, C), lambda i: (i, 0)),
            pl.BlockSpec((R, CL), lambda i: (i, 0)),
            pl.BlockSpec((2, R, C), lambda i: (0, i, 0)),
            pl.BlockSpec((2, R, CL), lambda i: (0, i, 0)),
        ],
        out_specs=pl.BlockSpec((R, C), lambda i: (i, 0)),
        out_shape=jax.ShapeDtypeStruct((N, C), jnp.float32),
    )(wv, h, gsum, gcnt, nsum, ncnt)


def kernel(x, w, W1, b1, W2, b2, graph_edge_index, hg_vertex, hg_edge):
    src = graph_edge_index[0]
    dst = graph_edge_index[1]
    ew = jnp.exp(w)
    wv = (ew / jnp.sum(ew)).reshape(1, 2)
    z128 = jnp.zeros((N // NSUB, C), jnp.float32)
    z16 = jnp.zeros((N // NSUB, CL), jnp.float32)
    ones = jnp.ones((K, CL), jnp.float32)

    gidx2 = jnp.stack([src.reshape(-1, K), dst.reshape(-1, K)], axis=1)
    hidx2 = jnp.stack([hg_vertex.reshape(-1, K), hg_edge.reshape(-1, K)], axis=1)
    eidx2 = jnp.stack([hg_edge.reshape(-1, K), hg_vertex.reshape(-1, K)], axis=1)

    h = _mlp(x, W1, b1.reshape(1, -1), W2, b2.reshape(1, -1))
    gsum, gcnt, esum, ecnt = _scA(h, gidx2, hidx2, z128, z16, ones)
    e_feat = _ecomb(esum, ecnt)
    nsum, ncnt = _scB(e_feat, eidx2, z128, z16, ones)
    return _final(wv, h, gsum, gcnt, nsum, ncnt)
